# Initial kernel scaffold; baseline (speedup 1.0000x reference)
import jax
import jax.numpy as jnp
from jax.experimental import pallas as pl


def kernel(indices, freq, u):
    raise NotImplementedError("write your pallas kernel here")


# speedup vs baseline: 133.3954x; 133.3954x over previous
import dataclasses
import functools

import jax
import jax.numpy as jnp
from jax import lax
from jax.experimental import pallas as pl
from jax.experimental.pallas import tpu as pltpu
from jax.experimental.pallas import tpu_sc as plsc

VOCAB = 100000
REPL_IDX = 1
NUM_CORES = 2
NUM_SUBCORES = 16
NUM_WORKERS = NUM_CORES * NUM_SUBCORES
LANES = 16


def _token_dropout_body(idx_hbm, freq_hbm, u_hbm, out_hbm,
                        freq_v, idx_v, u_v, per_worker, chunk):
    wid = lax.axis_index("s") * NUM_CORES + lax.axis_index("c")
    pltpu.sync_copy(freq_hbm, freq_v)
    base_w = wid * per_worker

    @pl.loop(0, per_worker // chunk)
    def _(ci):
        base = base_w + ci * chunk
        pltpu.sync_copy(idx_hbm.at[pl.ds(base, chunk)], idx_v)
        pltpu.sync_copy(u_hbm.at[pl.ds(base, chunk)], u_v)

        @pl.loop(0, chunk, step=LANES)
        def _(i):
            sl = pl.ds(i, LANES)
            iv = idx_v[sl]
            g = plsc.load_gather(freq_v, [iv])
            uv = u_v[sl]
            idx_v[sl] = jnp.where(uv < g, jnp.int32(REPL_IDX), iv)

        pltpu.sync_copy(idx_v, out_hbm.at[pl.ds(base, chunk)])


def kernel(indices, freq, u):
    shape = indices.shape
    n = indices.size
    per_worker = n // NUM_WORKERS
    chunk = 6400
    assert per_worker * NUM_WORKERS == n and per_worker % chunk == 0

    idx_flat = indices.reshape(n)
    u_flat = u.reshape(n)

    mesh = plsc.VectorSubcoreMesh(core_axis_name="c", subcore_axis_name="s")
    body = functools.partial(_token_dropout_body,
                             per_worker=per_worker, chunk=chunk)
    cp = pltpu.CompilerParams()
    if "needs_layout_passes" in pltpu.CompilerParams.__dataclass_fields__:
        cp = dataclasses.replace(cp, needs_layout_passes=False)
    run = pl.kernel(
        body,
        out_type=jax.ShapeDtypeStruct((n,), jnp.int32),
        mesh=mesh,
        compiler_params=cp,
        scratch_types=[
            pltpu.VMEM((VOCAB,), jnp.float32),
            pltpu.VMEM((chunk,), jnp.int32),
            pltpu.VMEM((chunk,), jnp.float32),
        ],
    )
    out = run(idx_flat, freq, u_flat)
    return out.reshape(shape)


# speedup vs baseline: 171.4944x; 1.2856x over previous
import dataclasses
import functools

import jax
import jax.numpy as jnp
from jax import lax
from jax.experimental import pallas as pl
from jax.experimental.pallas import tpu as pltpu
from jax.experimental.pallas import tpu_sc as plsc

VOCAB = 100000
REPL_IDX = 1
NUM_CORES = 2
NUM_SUBCORES = 16
NUM_WORKERS = NUM_CORES * NUM_SUBCORES
LANES = 16


def _token_dropout_body(idx_hbm, freq_hbm, u_hbm, out_hbm,
                        freq_v, idx_v, u_v, res_v, per_worker, chunk):
    wid = lax.axis_index("s") * NUM_CORES + lax.axis_index("c")
    pltpu.sync_copy(freq_hbm, freq_v)
    base_w = wid * per_worker

    @pl.loop(0, per_worker // chunk)
    def _(ci):
        base = base_w + ci * chunk
        pltpu.sync_copy(idx_hbm.at[pl.ds(base, chunk)], idx_v)
        pltpu.sync_copy(u_hbm.at[pl.ds(base, chunk)], u_v)

        @plsc.parallel_loop(0, chunk, step=LANES, unroll=8)
        def _(i):
            sl = pl.ds(i, LANES)
            iv = idx_v[sl]
            g = plsc.load_gather(freq_v, [iv])
            uv = u_v[sl]
            res_v[sl] = jnp.where(uv < g, jnp.int32(REPL_IDX), iv)

        pltpu.sync_copy(res_v, out_hbm.at[pl.ds(base, chunk)])


def kernel(indices, freq, u):
    shape = indices.shape
    n = indices.size
    per_worker = n // NUM_WORKERS
    chunk = 6400
    assert per_worker * NUM_WORKERS == n and per_worker % chunk == 0

    idx_flat = indices.reshape(n)
    u_flat = u.reshape(n)

    mesh = plsc.VectorSubcoreMesh(core_axis_name="c", subcore_axis_name="s")
    body = functools.partial(_token_dropout_body,
                             per_worker=per_worker, chunk=chunk)
    cp = pltpu.CompilerParams()
    if "needs_layout_passes" in pltpu.CompilerParams.__dataclass_fields__:
        cp = dataclasses.replace(cp, needs_layout_passes=False)
    run = pl.kernel(
        body,
        out_type=jax.ShapeDtypeStruct((n,), jnp.int32),
        mesh=mesh,
        compiler_params=cp,
        scratch_types=[
            pltpu.VMEM((VOCAB,), jnp.float32),
            pltpu.VMEM((chunk,), jnp.int32),
            pltpu.VMEM((chunk,), jnp.float32),
            pltpu.VMEM((chunk,), jnp.int32),
        ],
    )
    out = run(idx_flat, freq, u_flat)
    return out.reshape(shape)


# speedup vs baseline: 252.8802x; 1.4746x over previous
import dataclasses
import functools

import jax
import jax.numpy as jnp
from jax import lax
from jax.experimental import pallas as pl
from jax.experimental.pallas import tpu as pltpu
from jax.experimental.pallas import tpu_sc as plsc

VOCAB = 100000
REPL_IDX = 1
NUM_CORES = 2
NUM_SUBCORES = 16
NUM_WORKERS = NUM_CORES * NUM_SUBCORES
LANES = 16


def _token_dropout_body(idx_hbm, freq_hbm, u_hbm, out_hbm,
                        freq_v, idx_v, u_v, res_v, rows_w, rows_c, cols):
    wid = lax.axis_index("s") * NUM_CORES + lax.axis_index("c")
    pltpu.sync_copy(freq_hbm, freq_v)
    row0_w = wid * rows_w

    col_starts = list(range(0, cols - LANES + 1, LANES))
    if col_starts[-1] != cols - LANES:
        col_starts.append(cols - LANES)

    @pl.loop(0, rows_w // rows_c)
    def _(ci):
        row0 = row0_w + ci * rows_c
        rs = pl.ds(row0, rows_c)
        pltpu.sync_copy(idx_hbm.at[rs, :], idx_v)
        pltpu.sync_copy(u_hbm.at[rs, :], u_v)

        @plsc.parallel_loop(0, rows_c, step=1, unroll=2)
        def _(r):
            for c in col_starts:
                sl = (r, pl.ds(c, LANES))
                iv = idx_v[sl]
                g = plsc.load_gather(freq_v, [iv])
                uv = u_v[sl]
                res_v[sl] = jnp.where(uv < g, jnp.int32(REPL_IDX), iv)

        pltpu.sync_copy(res_v, out_hbm.at[rs, :])


def kernel(indices, freq, u):
    rows, cols = indices.shape
    rows_w = rows // NUM_WORKERS
    rows_c = 32
    assert rows_w * NUM_WORKERS == rows and rows_w % rows_c == 0

    mesh = plsc.VectorSubcoreMesh(core_axis_name="c", subcore_axis_name="s")
    body = functools.partial(_token_dropout_body,
                             rows_w=rows_w, rows_c=rows_c, cols=cols)
    cp = pltpu.CompilerParams(use_tc_tiling_on_sc=True)
    if "needs_layout_passes" in pltpu.CompilerParams.__dataclass_fields__:
        cp = dataclasses.replace(cp, needs_layout_passes=False)
    run = pl.kernel(
        body,
        out_type=jax.ShapeDtypeStruct((rows, cols), jnp.int32),
        mesh=mesh,
        compiler_params=cp,
        scratch_types=[
            pltpu.VMEM((VOCAB,), jnp.float32),
            pltpu.VMEM((rows_c, cols), jnp.int32),
            pltpu.VMEM((rows_c, cols), jnp.float32),
            pltpu.VMEM((rows_c, cols), jnp.int32),
        ],
    )
    return run(indices, freq, u)


# speedup vs baseline: 349.8105x; 1.3833x over previous
import dataclasses
import functools

import jax
import jax.numpy as jnp
from jax import lax
from jax.experimental import pallas as pl
from jax.experimental.pallas import tpu as pltpu
from jax.experimental.pallas import tpu_sc as plsc

VOCAB = 100000
REPL_IDX = 1
NUM_CORES = 2
NUM_SUBCORES = 16
NUM_WORKERS = NUM_CORES * NUM_SUBCORES
LANES = 16
TILE_R = 8
ITEM_C = 512


def _token_dropout_body(idx_hbm, freq_hbm, u_hbm, out_hbm,
                        freq_v, idx_v, u_v, res_v, n_cc, items_per_worker):
    wid = lax.axis_index("s") * NUM_CORES + lax.axis_index("c")
    pltpu.sync_copy(freq_hbm, freq_v)

    @pl.loop(0, items_per_worker)
    def _(k):
        item = wid * items_per_worker + k
        tr = item // n_cc
        cc = lax.rem(item, n_cc)
        rs = pl.ds(tr * TILE_R, TILE_R)
        cs = pl.ds(cc * ITEM_C, ITEM_C)
        pltpu.sync_copy(idx_hbm.at[rs, cs], idx_v)
        pltpu.sync_copy(u_hbm.at[rs, cs], u_v)

        @pl.loop(0, TILE_R)
        def _(r):
            @plsc.parallel_loop(0, ITEM_C, step=LANES, unroll=4)
            def _(c):
                sl = (r, pl.ds(c, LANES))
                iv = idx_v[sl]
                g = plsc.load_gather(freq_v, [iv])
                uv = u_v[sl]
                res_v[sl] = jnp.where(uv < g, jnp.int32(REPL_IDX), iv)

        pltpu.sync_copy(res_v, out_hbm.at[rs, cs])


def kernel(indices, freq, u):
    rows, cols = indices.shape
    rt, ct = cols, rows
    n_items = (rt // TILE_R) * (ct // ITEM_C)
    assert rt % TILE_R == 0 and ct % ITEM_C == 0 and n_items % NUM_WORKERS == 0
    items_per_worker = n_items // NUM_WORKERS
    n_cc = ct // ITEM_C

    mesh = plsc.VectorSubcoreMesh(core_axis_name="c", subcore_axis_name="s")
    body = functools.partial(_token_dropout_body,
                             n_cc=n_cc, items_per_worker=items_per_worker)
    cp = pltpu.CompilerParams(use_tc_tiling_on_sc=True)
    if "needs_layout_passes" in pltpu.CompilerParams.__dataclass_fields__:
        cp = dataclasses.replace(cp, needs_layout_passes=False)
    run = pl.kernel(
        body,
        out_type=jax.ShapeDtypeStruct((rt, ct), jnp.int32),
        mesh=mesh,
        compiler_params=cp,
        scratch_types=[
            pltpu.VMEM((VOCAB,), jnp.float32),
            pltpu.VMEM((TILE_R, ITEM_C), jnp.int32),
            pltpu.VMEM((TILE_R, ITEM_C), jnp.float32),
            pltpu.VMEM((TILE_R, ITEM_C), jnp.int32),
        ],
    )
    return run(indices.T, freq, u.T).T


# speedup vs baseline: 604.1006x; 1.7269x over previous
import dataclasses
import functools

import jax
import jax.numpy as jnp
from jax import lax
from jax.experimental import pallas as pl
from jax.experimental.pallas import tpu as pltpu
from jax.experimental.pallas import tpu_sc as plsc

VOCAB = 100000
REPL_IDX = 1
NUM_CORES = 2
NUM_SUBCORES = 16
NUM_WORKERS = NUM_CORES * NUM_SUBCORES
LANES = 16
TILE_R = 8
ITEM_C = 512


def _token_dropout_body(idx_hbm, freq_hbm, u_hbm, out_hbm,
                        freq_v, n_tr, n_cc):
    pltpu.sync_copy(freq_hbm, freq_v)

    def block_body(idx_v, u_v, res_v):
        @pl.loop(0, TILE_R)
        def _(r):
            @plsc.parallel_loop(0, ITEM_C, step=LANES, unroll=4)
            def _(c):
                sl = (r, pl.ds(c, LANES))
                iv = idx_v[sl]
                g = plsc.load_gather(freq_v, [iv])
                uv = u_v[sl]
                res_v[sl] = jnp.where(uv < g, jnp.int32(REPL_IDX), iv)

    blk = pl.BlockSpec((TILE_R, ITEM_C), index_map=lambda i, j: (i, j))
    pltpu.emit_pipeline(
        block_body,
        grid=(n_tr, n_cc),
        in_specs=[blk, blk],
        out_specs=[blk],
        core_axis_name=("c", "s"),
        dimension_semantics=(pltpu.PARALLEL, pltpu.PARALLEL),
    )(idx_hbm, u_hbm, out_hbm)


def kernel(indices, freq, u):
    rows, cols = indices.shape
    rt, ct = cols, rows
    assert rt % TILE_R == 0 and ct % ITEM_C == 0
    n_tr, n_cc = rt // TILE_R, ct // ITEM_C

    mesh = plsc.VectorSubcoreMesh(core_axis_name="c", subcore_axis_name="s")
    body = functools.partial(_token_dropout_body, n_tr=n_tr, n_cc=n_cc)
    cp = pltpu.CompilerParams(use_tc_tiling_on_sc=True)
    if "needs_layout_passes" in pltpu.CompilerParams.__dataclass_fields__:
        cp = dataclasses.replace(cp, needs_layout_passes=False)
    run = pl.kernel(
        body,
        out_type=jax.ShapeDtypeStruct((rt, ct), jnp.int32),
        mesh=mesh,
        compiler_params=cp,
        scratch_types=[
            pltpu.VMEM((VOCAB,), jnp.float32),
        ],
    )
    return run(indices.T, freq, u.T).T
